# Initial kernel scaffold; baseline (speedup 1.0000x reference)
import jax
import jax.numpy as jnp
from jax.experimental import pallas as pl


def kernel(x, W_backbone, b_backbone, W_gate, b_gate, W_experts, b_experts):
    raise NotImplementedError("write your pallas kernel here")


# speedup vs baseline: 2.4529x; 2.4529x over previous
import functools

import jax
import jax.numpy as jnp
from jax.experimental import pallas as pl
from jax.experimental.pallas import tpu as pltpu

N = 8192
D = 2048
H = 2048
E = 8
K = 2
C = 1000

GATE_TB = 256
FEAT_TB = 512
EXP_TB = 512
CP = 1024
EXP_CB = 256


def _gate_body(x_ref, wg_ref, bg_ref, gw_ref, idx_ref):
    x = x_ref[...]
    logits = jnp.dot(x, wg_ref[...], preferred_element_type=jnp.float32)
    logits = logits + bg_ref[...]
    m = jnp.max(logits, axis=1, keepdims=True)
    ex = jnp.exp(logits - m)
    probs = ex / jnp.sum(ex, axis=1, keepdims=True)
    iota_e = jax.lax.broadcasted_iota(jnp.int32, probs.shape, 1)
    m1 = jnp.max(probs, axis=1, keepdims=True)
    a1 = jnp.min(jnp.where(probs == m1, iota_e, E), axis=1, keepdims=True)
    masked = jnp.where(iota_e == a1, -jnp.inf, probs)
    m2 = jnp.max(masked, axis=1, keepdims=True)
    a2 = jnp.min(jnp.where(masked == m2, iota_e, E), axis=1, keepdims=True)
    gw = jnp.where(iota_e == a1, m1, 0.0) + jnp.where(iota_e == a2, m2, 0.0)
    gw_ref[...] = gw
    idx_ref[...] = jnp.concatenate([a1, a2], axis=1)


def _backbone_body(x_ref, wb_ref, bb_ref, feat_ref):
    acc = jnp.dot(x_ref[...], wb_ref[...], preferred_element_type=jnp.float32)
    feat_ref[...] = jnp.maximum(acc + bb_ref[...], 0.0)


def _experts_body(feat_ref, gw_ref, we_ref, be_ref, out_ref):
    f = feat_ref[...]
    gw = gw_ref[...]
    acc = jnp.zeros(out_ref.shape, dtype=jnp.float32)
    for e in range(E):
        pe = jnp.dot(f, we_ref[e], preferred_element_type=jnp.float32)
        pe = pe + be_ref[e]
        acc = acc + gw[:, e:e + 1] * pe
    out_ref[...] = acc


def kernel(x, W_backbone, b_backbone, W_gate, b_gate, W_experts, b_experts):
    bg2 = b_gate.reshape(1, E)
    bb2 = b_backbone.reshape(1, H)

    gate_weights, top_k_indices = pl.pallas_call(
        _gate_body,
        grid=(N // GATE_TB,),
        in_specs=[
            pl.BlockSpec((GATE_TB, D), lambda i: (i, 0)),
            pl.BlockSpec((D, E), lambda i: (0, 0)),
            pl.BlockSpec((1, E), lambda i: (0, 0)),
        ],
        out_specs=[
            pl.BlockSpec((GATE_TB, E), lambda i: (i, 0)),
            pl.BlockSpec((GATE_TB, K), lambda i: (i, 0)),
        ],
        out_shape=[
            jax.ShapeDtypeStruct((N, E), jnp.float32),
            jax.ShapeDtypeStruct((N, K), jnp.int32),
        ],
        compiler_params=pltpu.CompilerParams(
            dimension_semantics=("parallel",)),
    )(x, W_gate, bg2)

    features = pl.pallas_call(
        _backbone_body,
        grid=(N // FEAT_TB,),
        in_specs=[
            pl.BlockSpec((FEAT_TB, D), lambda i: (i, 0)),
            pl.BlockSpec((D, H), lambda i: (0, 0)),
            pl.BlockSpec((1, H), lambda i: (0, 0)),
        ],
        out_specs=pl.BlockSpec((FEAT_TB, H), lambda i: (i, 0)),
        out_shape=jax.ShapeDtypeStruct((N, H), jnp.float32),
        compiler_params=pltpu.CompilerParams(
            dimension_semantics=("parallel",)),
    )(x, W_backbone, bb2)

    we_pad = jnp.pad(W_experts, ((0, 0), (0, 0), (0, CP - C)))
    be_pad = jnp.pad(b_experts, ((0, 0), (0, CP - C)))

    combined = pl.pallas_call(
        _experts_body,
        grid=(CP // EXP_CB, N // EXP_TB),
        in_specs=[
            pl.BlockSpec((EXP_TB, H), lambda c, t: (t, 0)),
            pl.BlockSpec((EXP_TB, E), lambda c, t: (t, 0)),
            pl.BlockSpec((E, H, EXP_CB), lambda c, t: (0, 0, c)),
            pl.BlockSpec((E, EXP_CB), lambda c, t: (0, c)),
        ],
        out_specs=pl.BlockSpec((EXP_TB, EXP_CB), lambda c, t: (t, c)),
        out_shape=jax.ShapeDtypeStruct((N, CP), jnp.float32),
        compiler_params=pltpu.CompilerParams(
            dimension_semantics=("arbitrary", "parallel")),
    )(features, gate_weights, we_pad, be_pad)

    return (combined[:, :C], gate_weights, top_k_indices)
